# Initial kernel scaffold; baseline (speedup 1.0000x reference)
import jax
import jax.numpy as jnp
from jax.experimental import pallas as pl


def kernel(users, adjacent_items, items_pool, items_weight, user_preference, item_preference, graph_indices, graph_values):
    raise NotImplementedError("write your pallas kernel here")


# speedup vs baseline: 6.7149x; 6.7149x over previous
import jax
import jax.numpy as jnp
from jax import lax
from jax.experimental import pallas as pl
from jax.experimental.pallas import tpu as pltpu
from jax.experimental.pallas import tpu_sc as plsc

NU = 25000
NN = 50000
H = 32
NLAYERS = 3
NB = 4096
NE = 800000

NC = 2
NS = 16
LANES = 16

NNP = 50176

CH = 128
RCH = 4
ROUND = CH * RCH
RPT = 98
EPT = RPT * ROUND
EPAD = NS * EPT
NT = NS * RPT
STRIPE = NNP // NS
BPT = NB // NS
BCH = BPT // CH


def _gcn_body(tab0, src2, dst3, val3, bidx, snap_a, snap_b, snap_c,
              urows, grows, acc, rows_v, sidx_v, didx_v, val_v,
              gsum, gtmp, bidx_v, sem):
    cid = lax.axis_index("c")
    wid = lax.axis_index("s")

    zero16 = jnp.zeros((LANES,), jnp.float32)
    tabs = [tab0, snap_a, snap_b, snap_c]

    for layer in range(NLAYERS):
        src_tab = tabs[layer]
        out_tab = tabs[layer + 1]

        @pl.loop(0, ROUND)
        def _(i):
            rows_v[i, 0:16] = zero16
            rows_v[i, 16:32] = zero16

        base = wid * STRIPE
        for q in range(STRIPE // ROUND):
            pltpu.sync_copy(rows_v, acc.at[pl.ds(base + q * ROUND, ROUND)])
        rem = STRIPE % ROUND
        if rem:
            pltpu.sync_copy(rows_v.at[pl.ds(0, rem)],
                            acc.at[pl.ds(base + STRIPE - rem, rem)])
        plsc.subcore_barrier()

        @pl.loop(0, RPT)
        def _(r):
            t = wid * RPT + r
            pltpu.sync_copy(src2.at[cid, t], sidx_v)
            pltpu.sync_copy(dst3.at[t], didx_v)
            pltpu.sync_copy(val3.at[t], val_v)
            descs = [
                pltpu.async_copy(src_tab.at[sidx_v.at[j]],
                                 rows_v.at[pl.ds(j * CH, CH)], sem)
                for j in range(RCH)
            ]
            for d in descs:
                d.wait()

            @pl.loop(0, ROUND // LANES)
            def _(k0):
                vals = val_v[k0 // (CH // LANES),
                             pl.ds((k0 % (CH // LANES)) * LANES, LANES)]
                for e in range(LANES):
                    vb = lax.broadcast_in_dim(vals[e], (LANES,), ())
                    row = k0 * LANES + e
                    rows_v[row, 0:16] = rows_v[row, 0:16] * vb
                    rows_v[row, 16:32] = rows_v[row, 16:32] * vb

            for j in range(RCH):
                pltpu.sync_copy(rows_v.at[pl.ds(j * CH, CH)],
                                acc.at[didx_v.at[j]], add=True)

        plsc.subcore_barrier()
        obase = cid * NNP + base
        for q in range(STRIPE // ROUND):
            pltpu.sync_copy(acc.at[pl.ds(base + q * ROUND, ROUND)],
                            out_tab.at[pl.ds(obase + q * ROUND, ROUND)])
        if rem:
            pltpu.sync_copy(acc.at[pl.ds(base + STRIPE - rem, rem)],
                            out_tab.at[pl.ds(obase + STRIPE - rem, rem)])
        plsc.subcore_barrier()

    for role in range(4):
        pltpu.sync_copy(bidx.at[cid, role, wid], bidx_v)
        for c in range(BCH):
            pltpu.async_copy(tabs[0].at[bidx_v.at[c]], gsum, sem).wait()
            for tab in tabs[1:]:
                pltpu.async_copy(tab.at[bidx_v.at[c]], gtmp, sem).wait()

                @pl.loop(0, CH)
                def _(e):
                    gsum[e, 0:16] = gsum[e, 0:16] + gtmp[e, 0:16]
                    gsum[e, 16:32] = gsum[e, 16:32] + gtmp[e, 16:32]

            off = wid * BPT + c * CH
            if role == 0:
                pltpu.sync_copy(gsum, urows.at[cid, pl.ds(off, CH)])
            else:
                pltpu.sync_copy(gsum, grows.at[cid, role - 1, pl.ds(off, CH)])


def _dot_body(u_ref, g_ref, out_ref):
    u = u_ref[...]
    g = g_ref[...]
    prod = u[:, None, :, :] * g
    out_ref[...] = jnp.sum(prod, axis=(0, 3)) * jnp.float32(0.0625)


@jax.jit
def _run(tab0, src2, dst3, val3, bidx):
    mesh = plsc.VectorSubcoreMesh(core_axis_name="c", subcore_axis_name="s")
    f32 = jnp.float32
    out_type = (
        jax.ShapeDtypeStruct((NC * NNP, H), f32),
        jax.ShapeDtypeStruct((NC * NNP, H), f32),
        jax.ShapeDtypeStruct((NC * NNP, H), f32),
        jax.ShapeDtypeStruct((NC, NB, H), f32),
        jax.ShapeDtypeStruct((NC, 3, NB, H), f32),
    )
    scratch = [
        pltpu.VMEM_SHARED((NNP, H), f32),
        pltpu.VMEM((ROUND, H), f32),
        pltpu.VMEM((RCH, CH), jnp.int32),
        pltpu.VMEM((RCH, CH), jnp.int32),
        pltpu.VMEM((RCH, CH), f32),
        pltpu.VMEM((CH, H), f32),
        pltpu.VMEM((CH, H), f32),
        pltpu.VMEM((BCH, CH), jnp.int32),
        pltpu.SemaphoreType.DMA,
    ]
    k = pl.kernel(_gcn_body, out_type=out_type, mesh=mesh,
                  scratch_types=scratch,
                  compiler_params=pltpu.CompilerParams(
                      use_tc_tiling_on_sc=False))
    _, _, _, urows, grows = k(tab0, src2, dst3, val3, bidx)
    scores = pl.pallas_call(
        _dot_body,
        out_shape=jax.ShapeDtypeStruct((3, NB), jnp.float32),
    )(urows, grows)
    return scores


def kernel(users, adjacent_items, items_pool, items_weight, user_preference,
           item_preference, graph_indices, graph_values):
    del items_weight
    pref = jnp.concatenate([user_preference, item_preference], axis=0)
    rowpad = ((0, NNP - NN), (0, 0))
    tab0 = jnp.concatenate(
        [jnp.pad(pref[:, :H], rowpad), jnp.pad(pref[:, H:], rowpad)], axis=0)

    src = graph_indices[1].astype(jnp.int32)
    dst = graph_indices[0].astype(jnp.int32)
    val = graph_values.astype(jnp.float32)
    pad = EPAD - NE
    src = jnp.pad(src, (0, pad))
    dst = jnp.pad(dst, (0, pad))
    val = jnp.pad(val, (0, pad))
    src2 = jnp.stack([src, src + NNP]).reshape(NC, NT, RCH, CH)
    dst3 = dst.reshape(NT, RCH, CH)
    val3 = val.reshape(NT, RCH, CH)

    u = users.astype(jnp.int32)
    a = adjacent_items.astype(jnp.int32) + NU
    w = items_pool[:, 0].astype(jnp.int32) + NU
    s = items_pool[:, 1].astype(jnp.int32) + NU
    roles = jnp.stack([u, a, w, s])
    bidx = jnp.stack([roles, roles + NNP])
    bidx = bidx.reshape(NC, 4, NS, BCH, CH)

    scores = _run(tab0, src2, dst3, val3, bidx)
    return (scores[0], scores[1], scores[2])


# speedup vs baseline: 8.0819x; 1.2036x over previous
import jax
import jax.numpy as jnp
from jax import lax
from jax.experimental import pallas as pl
from jax.experimental.pallas import tpu as pltpu
from jax.experimental.pallas import tpu_sc as plsc

NU = 25000
NN = 50000
H = 32
NLAYERS = 3
NB = 4096
NE = 800000

NC = 2
NS = 16
LANES = 16

NNP = 50176

CH = 128
RCH = 2
ROUND = CH * RCH
RPT = 196
EPT = RPT * ROUND
EPAD = NS * EPT
NT = NS * RPT
STRIPE = NNP // NS
BPT = NB // NS
BCH = BPT // CH


def _gcn_body(tab0, edat_h, val_h, bidx, snap_a, snap_b, snap_c,
              urows, grows, acc, rows2, edat, valv, bidx_v, gsem, ssem):
    cid = lax.axis_index("c")
    wid = lax.axis_index("s")

    zero16 = jnp.zeros((LANES,), jnp.float32)
    tabs = [tab0, snap_a, snap_b, snap_c]

    def fire_gather(src_tab, p):
        return [
            pltpu.async_copy(src_tab.at[edat.at[p, 0, j]],
                             rows2.at[p, pl.ds(j * CH, CH)], gsem)
            for j in range(RCH)
        ]

    def wait_gather(src_tab, p):
        for j in range(RCH):
            pltpu.make_async_copy(src_tab.at[edat.at[p, 0, j]],
                                  rows2.at[p, pl.ds(j * CH, CH)], gsem).wait()

    def fire_scatter(p):
        for j in range(RCH):
            pltpu.async_copy(rows2.at[p, pl.ds(j * CH, CH)],
                             acc.at[edat.at[p, 1, j]], ssem, add=True)

    def drain_scatter(p):
        for j in range(RCH):
            pltpu.make_async_copy(rows2.at[p, pl.ds(j * CH, CH)],
                                  acc.at[edat.at[p, 1, j]], ssem).wait()

    def scale_rows(p):
        @pl.loop(0, ROUND // LANES)
        def _(k0):
            vrow = k0 // (CH // LANES)
            vcol = (k0 % (CH // LANES)) * LANES
            vals = valv[p, vrow, pl.ds(vcol, LANES)]
            for e in range(LANES):
                vb = lax.broadcast_in_dim(vals[e], (LANES,), ())
                row = k0 * LANES + e
                rows2[p, row, 0:16] = rows2[p, row, 0:16] * vb
                rows2[p, row, 16:32] = rows2[p, row, 16:32] * vb

    for layer in range(NLAYERS):
        src_tab = tabs[layer]
        out_tab = tabs[layer + 1]

        @pl.loop(0, ROUND)
        def _(i):
            rows2[0, i, 0:16] = zero16
            rows2[0, i, 16:32] = zero16

        base = wid * STRIPE
        for q in range(STRIPE // ROUND):
            pltpu.sync_copy(rows2.at[0],
                            acc.at[pl.ds(base + q * ROUND, ROUND)])
        rem = STRIPE % ROUND
        if rem:
            pltpu.sync_copy(rows2.at[0, pl.ds(0, rem)],
                            acc.at[pl.ds(base + STRIPE - rem, rem)])
        plsc.subcore_barrier()

        tb = wid * RPT
        pltpu.sync_copy(edat_h.at[cid, tb], edat.at[0])
        pltpu.sync_copy(val_h.at[tb], valv.at[0])
        fire_gather(src_tab, 0)

        @pl.loop(0, RPT // 2)
        def _(rr):
            for p in (0, 1):
                r = rr * 2 + p
                q = 1 - p

                @pl.when(r + 1 < RPT)
                def _():
                    @pl.when(r >= 1)
                    def _():
                        drain_scatter(q)
                    pltpu.sync_copy(edat_h.at[cid, tb + r + 1], edat.at[q])
                    pltpu.sync_copy(val_h.at[tb + r + 1], valv.at[q])
                    fire_gather(src_tab, q)

                wait_gather(src_tab, p)
                scale_rows(p)
                fire_scatter(p)

        drain_scatter(0)
        drain_scatter(1)
        plsc.subcore_barrier()

        obase = cid * NNP + base
        for q in range(STRIPE // ROUND):
            pltpu.sync_copy(acc.at[pl.ds(base + q * ROUND, ROUND)],
                            out_tab.at[pl.ds(obase + q * ROUND, ROUND)])
        if rem:
            pltpu.sync_copy(acc.at[pl.ds(base + STRIPE - rem, rem)],
                            out_tab.at[pl.ds(obase + STRIPE - rem, rem)])
        plsc.subcore_barrier()

    gsum = rows2.at[0, pl.ds(0, CH)]
    gtmp = rows2.at[0, pl.ds(CH, CH)]
    for role in range(4):
        pltpu.sync_copy(bidx.at[cid, role, wid], bidx_v)
        for c in range(BCH):
            pltpu.async_copy(tabs[0].at[bidx_v.at[c]], gsum, gsem).wait()
            for tab in tabs[1:]:
                pltpu.async_copy(tab.at[bidx_v.at[c]], gtmp, gsem).wait()

                @pl.loop(0, CH)
                def _(e):
                    rows2[0, e, 0:16] = rows2[0, e, 0:16] + rows2[0, CH + e, 0:16]
                    rows2[0, e, 16:32] = (rows2[0, e, 16:32]
                                          + rows2[0, CH + e, 16:32])

            off = wid * BPT + c * CH
            if role == 0:
                pltpu.sync_copy(gsum, urows.at[cid, pl.ds(off, CH)])
            else:
                pltpu.sync_copy(gsum, grows.at[cid, role - 1, pl.ds(off, CH)])


def _dot_body(u_ref, g_ref, out_ref):
    u = u_ref[...]
    g = g_ref[...]
    prod = u[:, None, :, :] * g
    out_ref[...] = jnp.sum(prod, axis=(0, 3)) * jnp.float32(0.0625)


@jax.jit
def _run(tab0, edat_h, val_h, bidx):
    mesh = plsc.VectorSubcoreMesh(core_axis_name="c", subcore_axis_name="s")
    f32 = jnp.float32
    out_type = (
        jax.ShapeDtypeStruct((NC * NNP, H), f32),
        jax.ShapeDtypeStruct((NC * NNP, H), f32),
        jax.ShapeDtypeStruct((NC * NNP, H), f32),
        jax.ShapeDtypeStruct((NC, NB, H), f32),
        jax.ShapeDtypeStruct((NC, 3, NB, H), f32),
    )
    scratch = [
        pltpu.VMEM_SHARED((NNP, H), f32),
        pltpu.VMEM((2, ROUND, H), f32),
        pltpu.VMEM((2, 2, RCH, CH), jnp.int32),
        pltpu.VMEM((2, RCH, CH), jnp.float32),
        pltpu.VMEM((BCH, CH), jnp.int32),
        pltpu.SemaphoreType.DMA,
        pltpu.SemaphoreType.DMA,
    ]
    k = pl.kernel(_gcn_body, out_type=out_type, mesh=mesh,
                  scratch_types=scratch,
                  compiler_params=pltpu.CompilerParams(
                      use_tc_tiling_on_sc=False))
    _, _, _, urows, grows = k(tab0, edat_h, val_h, bidx)
    scores = pl.pallas_call(
        _dot_body,
        out_shape=jax.ShapeDtypeStruct((3, NB), jnp.float32),
    )(urows, grows)
    return scores


def kernel(users, adjacent_items, items_pool, items_weight, user_preference,
           item_preference, graph_indices, graph_values):
    del items_weight
    pref = jnp.concatenate([user_preference, item_preference], axis=0)
    rowpad = ((0, NNP - NN), (0, 0))
    tab0 = jnp.concatenate(
        [jnp.pad(pref[:, :H], rowpad), jnp.pad(pref[:, H:], rowpad)], axis=0)

    src = graph_indices[1].astype(jnp.int32)
    dst = graph_indices[0].astype(jnp.int32)
    val = graph_values.astype(jnp.float32)
    pad = EPAD - NE
    src = jnp.pad(src, (0, pad))
    dst = jnp.pad(dst, (0, pad))
    val = jnp.pad(val, (0, pad))
    edat_h = jnp.stack([
        jnp.stack([src.reshape(NT, RCH, CH), dst.reshape(NT, RCH, CH)],
                  axis=1),
        jnp.stack([(src + NNP).reshape(NT, RCH, CH),
                   dst.reshape(NT, RCH, CH)], axis=1),
    ])
    val_h = val.reshape(NT, RCH, CH)

    u = users.astype(jnp.int32)
    a = adjacent_items.astype(jnp.int32) + NU
    w = items_pool[:, 0].astype(jnp.int32) + NU
    s = items_pool[:, 1].astype(jnp.int32) + NU
    roles = jnp.stack([u, a, w, s])
    bidx = jnp.stack([roles, roles + NNP])
    bidx = bidx.reshape(NC, 4, NS, BCH, CH)

    scores = _run(tab0, edat_h, val_h, bidx)
    return (scores[0], scores[1], scores[2])
